# baseline (speedup 1.0000x reference)
import functools

import jax
import jax.numpy as jnp
from jax import lax
from jax.experimental import pallas as pl
from jax.experimental.pallas import tpu as pltpu
from jax.experimental.pallas import tpu_sc as plsc

_KN = 1024
_X0 = -10.0
_DX = (10.0 - (-10.0)) / (_KN - 1)
_INV = 1.0 / (_DX + 1e-12)
_NW = 32
_L = 16
_RB = 8
_CB = 2048


def _spline_body(nrows, ncols, x_hbm, c0_hbm, c1_hbm, o_hbm, c0_v, c1_v, xb0,
                 xb1, xb2, xb3, ob0, ob1, si0, si1, si2, si3, so0, so1):
    c = lax.axis_index("c")
    s = lax.axis_index("s")
    wrow = (s * 2 + c) * (nrows // _NW)
    nch = (nrows // _NW) // _RB * (ncols // _CB)
    ncw = ncols // _CB
    pltpu.sync_copy(c0_hbm, c0_v)
    pltpu.sync_copy(c1_hbm, c1_v)
    xbufs = (xb0, xb1, xb2, xb3)
    obufs = (ob0, ob1)
    sin = (si0, si1, si2, si3)
    sout = (so0, so1)

    def slab(k):
        r0 = wrow + (k // ncw) * _RB
        c0_ = (k % ncw) * _CB
        return (pl.ds(r0, _RB), pl.ds(c0_, _CB))

    def start_in(k, b):
        pltpu.async_copy(x_hbm.at[slab(k)], xbufs[b], sin[b])

    def wait_in(b):
        pltpu.make_async_copy(x_hbm.at[slab(0)], xbufs[b], sin[b]).wait()

    def start_out(k, b):
        pltpu.async_copy(obufs[b], o_hbm.at[slab(k)], sout[b])

    def wait_out(b):
        pltpu.make_async_copy(obufs[b], o_hbm.at[slab(0)], sout[b]).wait()

    def compute(bi, bo):
        xb = xbufs[bi]
        ob = obufs[bo]

        @plsc.parallel_loop(0, _CB, step=_L, unroll=1)
        def vec(i):
            for r in range(_RB):
                xv = xb[r, pl.ds(i, _L)]
                u = (xv - _X0) * _INV
                uf = jnp.minimum(jnp.maximum(u, 0.0), float(_KN - 2))
                idx = uf.astype(jnp.int32)
                g0 = plsc.load_gather(c0_v, [idx])
                g1 = plsc.load_gather(c1_v, [idx])
                ob[r, pl.ds(i, _L)] = g0 + u * g1

    for b in range(4):
        start_in(b, b)

    @pl.loop(0, nch, step=4)
    def outer(k0):
        for b in range(4):
            k = k0 + b
            bo = b % 2

            wait_in(b)

            @pl.when(k >= 2)
            def _():
                wait_out(bo)

            compute(b, bo)
            start_out(k, bo)

            @pl.when(k + 4 < nch)
            def _():
                start_in(k + 4, b)

    wait_out(0)
    wait_out(1)


@jax.jit
def kernel(x, y):
    nrows, ncols = x.shape
    assert nrows % (_NW * _RB) == 0 and ncols % _CB == 0
    assert (nrows // _NW // _RB) * (ncols // _CB) % 4 == 0
    dy = jnp.diff(y)
    iarr = jnp.arange(_KN - 1, dtype=jnp.float32)
    c0 = jnp.pad(y[:-1] - iarr * dy, (0, 1))
    c1 = jnp.pad(dy, (0, 1))
    mesh = plsc.VectorSubcoreMesh(core_axis_name="c", subcore_axis_name="s")
    run = pl.kernel(
        functools.partial(_spline_body, nrows, ncols),
        out_type=jax.ShapeDtypeStruct((nrows, ncols), jnp.float32),
        mesh=mesh,
        scratch_types=[
            pltpu.VMEM((_KN,), jnp.float32),
            pltpu.VMEM((_KN,), jnp.float32),
            pltpu.VMEM((_RB, _CB), jnp.float32),
            pltpu.VMEM((_RB, _CB), jnp.float32),
            pltpu.VMEM((_RB, _CB), jnp.float32),
            pltpu.VMEM((_RB, _CB), jnp.float32),
            pltpu.VMEM((_RB, _CB), jnp.float32),
            pltpu.VMEM((_RB, _CB), jnp.float32),
            pltpu.SemaphoreType.DMA,
            pltpu.SemaphoreType.DMA,
            pltpu.SemaphoreType.DMA,
            pltpu.SemaphoreType.DMA,
            pltpu.SemaphoreType.DMA,
            pltpu.SemaphoreType.DMA,
        ],
        compiler_params=pltpu.CompilerParams(
            needs_layout_passes=False, use_tc_tiling_on_sc=True
        ),
    )
    return run(x, c0, c1)


# speedup vs baseline: 1.0487x; 1.0487x over previous
import functools

import jax
import jax.numpy as jnp
from jax import lax
from jax.experimental import pallas as pl
from jax.experimental.pallas import tpu as pltpu
from jax.experimental.pallas import tpu_sc as plsc

_KN = 1024
_X0 = -10.0
_DX = (10.0 - (-10.0)) / (_KN - 1)
_INV = 1.0 / (_DX + 1e-12)
_NW = 32
_L = 16
_RB = 8
_CB = 2048


def _spline_body(nrows, ncols, x_hbm, y_hbm, o_hbm, y_v, c0_v, c1_v, xb0, xb1,
                 ob0, ob1, si0, si1, so0, so1, sy):
    c = lax.axis_index("c")
    s = lax.axis_index("s")
    wrow = (s * 2 + c) * (nrows // _NW)
    nch = (nrows // _NW) // _RB * (ncols // _CB)
    ncw = ncols // _CB
    xbufs = (xb0, xb1)
    obufs = (ob0, ob1)
    sin = (si0, si1)
    sout = (so0, so1)

    def slab(k):
        r0 = wrow + (k // ncw) * _RB
        c0_ = (k % ncw) * _CB
        return (pl.ds(r0, _RB), pl.ds(c0_, _CB))

    def start_in(k, b):
        pltpu.async_copy(x_hbm.at[slab(k)], xbufs[b], sin[b])

    def wait_in(b):
        pltpu.make_async_copy(x_hbm.at[slab(0)], xbufs[b], sin[b]).wait()

    def start_out(k, b):
        pltpu.async_copy(obufs[b], o_hbm.at[slab(k)], sout[b])

    def wait_out(b):
        pltpu.make_async_copy(obufs[b], o_hbm.at[slab(0)], sout[b]).wait()

    start_in(0, 0)
    start_in(1, 1)

    pltpu.async_copy(y_hbm, y_v.at[pl.ds(0, _KN)], sy).wait()

    @plsc.parallel_loop(0, _KN, step=_L, unroll=4)
    def mktab(i):
        yl = y_v[pl.ds(i, _L)]
        yr = y_v[pl.ds(i + 1, _L)]
        dy = yr - yl
        if32 = (i + lax.iota(jnp.int32, _L)).astype(jnp.float32)
        c1_v[pl.ds(i, _L)] = dy
        c0_v[pl.ds(i, _L)] = yl - if32 * dy

    def compute(b):
        xb = xbufs[b]
        ob = obufs[b]

        @plsc.parallel_loop(0, _CB, step=_L, unroll=1)
        def vec(i):
            for r in range(_RB):
                xv = xb[r, pl.ds(i, _L)]
                u = (xv - _X0) * _INV
                uf = jnp.minimum(jnp.maximum(u, 0.0), float(_KN - 2))
                idx = uf.astype(jnp.int32)
                g0 = plsc.load_gather(c0_v, [idx])
                g1 = plsc.load_gather(c1_v, [idx])
                ob[r, pl.ds(i, _L)] = g0 + u * g1

    @pl.loop(0, nch, step=2)
    def outer(k0):
        for b in range(2):
            k = k0 + b

            wait_in(b)

            @pl.when(k >= 2)
            def _():
                wait_out(b)

            compute(b)
            start_out(k, b)

            @pl.when(k + 2 < nch)
            def _():
                start_in(k + 2, b)

    wait_out(0)
    wait_out(1)


@jax.jit
def kernel(x, y):
    nrows, ncols = x.shape
    assert nrows % (_NW * _RB) == 0 and ncols % _CB == 0
    mesh = plsc.VectorSubcoreMesh(core_axis_name="c", subcore_axis_name="s")
    run = pl.kernel(
        functools.partial(_spline_body, nrows, ncols),
        out_type=jax.ShapeDtypeStruct((nrows, ncols), jnp.float32),
        mesh=mesh,
        scratch_types=[
            pltpu.VMEM((_KN + _L,), jnp.float32),
            pltpu.VMEM((_KN,), jnp.float32),
            pltpu.VMEM((_KN,), jnp.float32),
            pltpu.VMEM((_RB, _CB), jnp.float32),
            pltpu.VMEM((_RB, _CB), jnp.float32),
            pltpu.VMEM((_RB, _CB), jnp.float32),
            pltpu.VMEM((_RB, _CB), jnp.float32),
            pltpu.SemaphoreType.DMA,
            pltpu.SemaphoreType.DMA,
            pltpu.SemaphoreType.DMA,
            pltpu.SemaphoreType.DMA,
            pltpu.SemaphoreType.DMA,
        ],
        compiler_params=pltpu.CompilerParams(
            needs_layout_passes=False, use_tc_tiling_on_sc=True
        ),
    )
    return run(x, y)
